# baseline (speedup 1.0000x reference)
import jax
import jax.numpy as jnp
from jax.experimental import pallas as pl
from jax.experimental.pallas import tpu as pltpu

_SIGMA = 0.1
_NCORES = 2
_NJ = 4
_CH = 256


def _round_up(v, m):
    return ((v + m - 1) // m) * m


def _make_kernel(bt, st, ncj):
    def _kern(x_ref, w_ref, nw_ref, b_ref, nb_ref, o_hbm,
              weff_ref, beff_ref, o_vm, out_sem):
        j = pl.program_id(1)
        base = pl.program_id(0) * bt + j * st

        @pl.when(j == 0)
        def _():
            weff_ref[...] = (w_ref[...]
                             + _SIGMA * nw_ref[...]).astype(jnp.bfloat16)
            beff_ref[...] = b_ref[...] + _SIGMA * nb_ref[...]

        for c in range(ncj):
            vsl = pl.ds(j * st + c * _CH, _CH)
            o_vm[vsl, :] = (
                jnp.dot(x_ref[pl.ds(c * _CH, _CH), :].astype(jnp.bfloat16),
                        weff_ref[...], preferred_element_type=jnp.float32)
                + beff_ref[...]
            )
            pltpu.make_async_copy(
                o_vm.at[vsl, :],
                o_hbm.at[pl.ds(base + c * _CH, _CH), :],
                out_sem.at[j * ncj + c]).start()

        @pl.when(j == pl.num_programs(1) - 1)
        def _():
            for jj in range(_NJ):
                for c in range(ncj):
                    row = jj * st + c * _CH
                    pltpu.make_async_copy(
                        o_vm.at[pl.ds(row, _CH), :],
                        o_hbm.at[pl.ds(pl.program_id(0) * bt + row, _CH), :],
                        out_sem.at[jj * ncj + c]).wait()

    return _kern


def kernel(x, w_t, bias2d, noise_w_t, noise_b2d):
    B, K = x.shape
    Kw, N = w_t.shape
    assert K == Kw

    bt = _round_up(B, _CH * _NJ * _NCORES) // _NCORES
    Bp = bt * _NCORES
    x_p = x if Bp == B else jnp.pad(x, ((0, Bp - B), (0, 0)))
    st = bt // _NJ
    ncj = st // _CH

    out = pl.pallas_call(
        _make_kernel(bt, st, ncj),
        grid=(_NCORES, _NJ),
        in_specs=[
            pl.BlockSpec((st, K), lambda i, j: (i * _NJ + j, 0)),
            pl.BlockSpec((K, N), lambda i, j: (0, 0)),
            pl.BlockSpec((K, N), lambda i, j: (0, 0)),
            pl.BlockSpec((1, N), lambda i, j: (0, 0)),
            pl.BlockSpec((1, N), lambda i, j: (0, 0)),
        ],
        out_specs=pl.BlockSpec(memory_space=pltpu.MemorySpace.HBM),
        out_shape=jax.ShapeDtypeStruct((Bp, N), jnp.float32),
        scratch_shapes=[
            pltpu.VMEM((K, N), jnp.bfloat16),
            pltpu.VMEM((1, N), jnp.float32),
            pltpu.VMEM((bt, N), jnp.float32),
            pltpu.SemaphoreType.DMA((_NJ * ncj,)),
        ],
        compiler_params=pltpu.CompilerParams(
            dimension_semantics=("parallel", "arbitrary"),
            vmem_limit_bytes=48 << 20,
        ),
    )(x_p, w_t, noise_w_t, bias2d, noise_b2d)

    return out if Bp == B else out[:B]


# speedup vs baseline: 1.1206x; 1.1206x over previous
import jax
import jax.numpy as jnp
from jax.experimental import pallas as pl
from jax.experimental.pallas import tpu as pltpu

_SIGMA = 0.1
_NCORES = 2
_CH = 256


def _round_up(v, m):
    return ((v + m - 1) // m) * m


def _make_kernel(bt, nc):
    def _kern(x_ref, w_ref, nw_ref, b_ref, nb_ref, o_hbm,
              weff_ref, beff_ref, o_vm, out_sem):
        base = pl.program_id(0) * bt

        weff_ref[...] = (w_ref[...] + _SIGMA * nw_ref[...]).astype(jnp.bfloat16)
        beff_ref[...] = b_ref[...] + _SIGMA * nb_ref[...]

        for c in range(nc):
            sl = pl.ds(c * _CH, _CH)
            o_vm[sl, :] = (
                jnp.dot(x_ref[sl, :].astype(jnp.bfloat16), weff_ref[...],
                        preferred_element_type=jnp.float32)
                + beff_ref[...]
            )
            pltpu.make_async_copy(
                o_vm.at[sl, :],
                o_hbm.at[pl.ds(base + c * _CH, _CH), :],
                out_sem.at[c]).start()

        for c in range(nc):
            pltpu.make_async_copy(
                o_vm.at[pl.ds(c * _CH, _CH), :],
                o_hbm.at[pl.ds(base + c * _CH, _CH), :],
                out_sem.at[c]).wait()

    return _kern


def kernel(x, w_t, bias2d, noise_w_t, noise_b2d):
    B, K = x.shape
    Kw, N = w_t.shape
    assert K == Kw

    bt = _round_up(B, _CH * _NCORES) // _NCORES
    Bp = bt * _NCORES
    x_p = x if Bp == B else jnp.pad(x, ((0, Bp - B), (0, 0)))
    nc = bt // _CH

    out = pl.pallas_call(
        _make_kernel(bt, nc),
        grid=(_NCORES,),
        in_specs=[
            pl.BlockSpec((bt, K), lambda i: (i, 0)),
            pl.BlockSpec((K, N), lambda i: (0, 0)),
            pl.BlockSpec((K, N), lambda i: (0, 0)),
            pl.BlockSpec((1, N), lambda i: (0, 0)),
            pl.BlockSpec((1, N), lambda i: (0, 0)),
        ],
        out_specs=pl.BlockSpec(memory_space=pltpu.MemorySpace.HBM),
        out_shape=jax.ShapeDtypeStruct((Bp, N), jnp.float32),
        scratch_shapes=[
            pltpu.VMEM((K, N), jnp.bfloat16),
            pltpu.VMEM((1, N), jnp.float32),
            pltpu.VMEM((bt, N), jnp.float32),
            pltpu.SemaphoreType.DMA((nc,)),
        ],
        compiler_params=pltpu.CompilerParams(
            dimension_semantics=("parallel",),
            vmem_limit_bytes=48 << 20,
        ),
    )(x_p, w_t, noise_w_t, bias2d, noise_b2d)

    return out if Bp == B else out[:B]
